# baseline (speedup 1.0000x reference)
import functools

import jax
import jax.numpy as jnp
import numpy as np
from jax.experimental import pallas as pl
from jax.experimental.pallas import tpu as pltpu

_N = 10000
_CIN = 128
_COUT = 64

_OFF2_NP = np.array([[a, b, c] for a in (0, 1) for b in (0, 1) for c in (0, 1)], dtype=np.int32)
_OFF3_NP = np.array([[a, b, c] for a in (-1, 0, 1) for b in (-1, 0, 1) for c in (-1, 0, 1)], dtype=np.int32)


def _lin(c, S):
    return (c[:, 0] * S + c[:, 1]) * S + c[:, 2]


def _leaky(f, s):
    return jnp.where(f >= 0, f, s * f)


def _bn(f, g, b, eps=1e-4):
    mu = jnp.mean(f, axis=0)
    var = jnp.var(f, axis=0)
    return (f - mu) / jnp.sqrt(var + eps) * g + b


def _neighbor_rows(coords, S):
    n = coords.shape[0]
    ids = _lin(coords, S)
    lut = jnp.full((S * S * S,), -1, dtype=jnp.int32).at[ids].set(jnp.arange(n, dtype=jnp.int32))
    nb = coords[None, :, :] + jnp.asarray(_OFF3_NP)[:, None, :]
    valid = jnp.all((nb >= 0) & (nb < S), axis=-1)
    nbid = (nb[..., 0] * S + nb[..., 1]) * S + nb[..., 2]
    nbid = jnp.clip(nbid, 0, S * S * S - 1)
    return jnp.where(valid, lut[nbid], -1)


def _subm_conv(feat, rows, W):
    out = jnp.zeros((feat.shape[0], W.shape[2]), dtype=feat.dtype)
    for k in range(27):
        r = rows[k]
        m = (r >= 0).astype(feat.dtype)
        f = feat[jnp.maximum(r, 0)] * m[:, None]
        out = out + f @ W[k]
    return out


def _final_kernel(h_ref, res_ref, o_ref):
    h = h_ref[...]
    r = res_ref[...]
    s = h + r
    o_ref[...] = jnp.where(s >= 0, s, 0.333 * s)


def _final_leaky(h, res):
    n = h.shape[0]
    blk = 4000
    return pl.pallas_call(
        _final_kernel,
        out_shape=jax.ShapeDtypeStruct(h.shape, h.dtype),
        grid=(n // blk,),
        in_specs=[
            pl.BlockSpec((blk, _COUT), lambda i: (i, 0)),
            pl.BlockSpec((blk, _COUT), lambda i: (i, 0)),
        ],
        out_specs=pl.BlockSpec((blk, _COUT), lambda i: (i, 0)),
    )(h, res)


def kernel(x, skip_features, cords, skip_cords, W_up, W1, g1, b1, W2, g2, b2, W3, g3, b3, spatial_size):
    S_t = 2 * spatial_size
    S = 2 * 64
    out_coords = (2 * cords[None, :, :] + jnp.asarray(_OFF2_NP)[:, None, :]).reshape(-1, 3)
    feat = jnp.einsum('nc,kcd->knd', x, W_up).reshape(-1, _COUT)
    id_x = _lin(out_coords, S_t)
    id_s = _lin(skip_cords, S_t)
    lut_s = jnp.full((S * S * S,), -1, dtype=jnp.int32).at[id_s].set(
        jnp.arange(skip_cords.shape[0], dtype=jnp.int32))
    r = lut_s[id_x]
    matched = r >= 0
    feat = jnp.where(matched[:, None], feat + skip_features[jnp.maximum(r, 0)], feat)
    rows = _neighbor_rows(out_coords, S)
    h = _leaky(_bn(_subm_conv(feat, rows, W1), g1, b1), 0.05)
    res = h
    h = _leaky(_bn(_subm_conv(h, rows, W2), g2, b2), 0.05)
    h = _bn(_subm_conv(h, rows, W3), g3, b3)
    return _final_leaky(h, res)


# speedup vs baseline: 1.4871x; 1.4871x over previous
import functools

import jax
import jax.numpy as jnp
import numpy as np
from jax import lax
from jax.experimental import pallas as pl
from jax.experimental.pallas import tpu as pltpu
from jax.experimental.pallas import tpu_sc as plsc

_N = 10000
_NOUT = 80000
_NPAD = 81920
_M = 150000
_MPAD = 150016
_CIN = 128
_COUT = 64
_S2 = 128
_BLK = 1000
_NB = 80
_TAP_ROWS = 81000
_ZROW = _NOUT
_EPS = 1e-4

_OFF2_NP = np.array([[a, b, c] for a in (0, 1) for b in (0, 1) for c in (0, 1)], dtype=np.int32)
_OFF3_NP = np.array([[a, b, c] for a in (-1, 0, 1) for b in (-1, 0, 1) for c in (-1, 0, 1)], dtype=np.int32)

_MESH = plsc.VectorSubcoreMesh(core_axis_name="c", subcore_axis_name="s")


def _up_body(x_ref, w_ref, o_ref):
    o_ref[0] = jnp.dot(x_ref[...], w_ref[0], preferred_element_type=jnp.float32)


def _upsample(x, w_up):
    return pl.pallas_call(
        _up_body,
        grid=(8, 10),
        in_specs=[
            pl.BlockSpec((_BLK, _CIN), lambda k, j: (j, 0)),
            pl.BlockSpec((1, _CIN, _COUT), lambda k, j: (k, 0, 0)),
        ],
        out_specs=pl.BlockSpec((1, _BLK, _COUT), lambda k, j: (k, j, 0)),
        out_shape=jax.ShapeDtypeStruct((8, _N, _COUT), jnp.float32),
    )(x, w_up)


def _p1_body(f_ref, sg_ref, w_ref, o_ref):
    b = pl.program_id(0)

    @pl.when(b == _NB)
    def _():
        o_ref[...] = jnp.zeros((_BLK, _COUT), jnp.float32)

    @pl.when(b < _NB)
    def _():
        h = f_ref[...] + sg_ref[...]
        o_ref[...] = jnp.dot(h, w_ref[0], preferred_element_type=jnp.float32)


def _pmid_body(s_ref, st_ref, gb_ref, w_ref, o_ref):
    b = pl.program_id(0)

    @pl.when(b == _NB)
    def _():
        o_ref[...] = jnp.zeros((_BLK, _COUT), jnp.float32)

    @pl.when(b < _NB)
    def _():
        st = st_ref[...]
        gb = gb_ref[...]
        mu = st[0:1, :] / float(_NOUT)
        var = st[1:2, :] / float(_NOUT) - mu * mu
        a = gb[0:1, :] * lax.rsqrt(var + _EPS)
        c = gb[1:2, :] - mu * a
        h = s_ref[...] * a + c
        h = jnp.where(h >= 0, h, 0.05 * h)
        o_ref[...] = jnp.dot(h, w_ref[0], preferred_element_type=jnp.float32)


def _p_table_first(feat_pad, skipg, w):
    return pl.pallas_call(
        _p1_body,
        grid=(_NB + 1, 27),
        in_specs=[
            pl.BlockSpec((_BLK, _COUT), lambda b, k: (b, 0)),
            pl.BlockSpec((_BLK, _COUT), lambda b, k: (b, 0)),
            pl.BlockSpec((1, _COUT, _COUT), lambda b, k: (k, 0, 0)),
        ],
        out_specs=pl.BlockSpec((_BLK, _COUT), lambda b, k: (k * (_NB + 1) + b, 0)),
        out_shape=jax.ShapeDtypeStruct((27 * _TAP_ROWS, _COUT), jnp.float32),
    )(feat_pad, skipg, w)


def _p_table_mid(s_prev, stats, gb, w):
    return pl.pallas_call(
        _pmid_body,
        grid=(_NB + 1, 27),
        in_specs=[
            pl.BlockSpec((_BLK, _COUT), lambda b, k: (b, 0)),
            pl.BlockSpec((8, _COUT), lambda b, k: (0, 0)),
            pl.BlockSpec((8, _COUT), lambda b, k: (0, 0)),
            pl.BlockSpec((1, _COUT, _COUT), lambda b, k: (k, 0, 0)),
        ],
        out_specs=pl.BlockSpec((_BLK, _COUT), lambda b, k: (k * (_NB + 1) + b, 0)),
        out_shape=jax.ShapeDtypeStruct((27 * _TAP_ROWS, _COUT), jnp.float32),
    )(s_prev, stats, gb, w)


def _stats_body(s_ref, o_ref):
    i = pl.program_id(0)

    @pl.when(i == 0)
    def _():
        o_ref[...] = jnp.zeros((8, _COUT), jnp.float32)

    x = s_ref[...]
    o_ref[0:1, :] += jnp.sum(x, axis=0, keepdims=True)
    o_ref[1:2, :] += jnp.sum(x * x, axis=0, keepdims=True)


def _stats(s):
    return pl.pallas_call(
        _stats_body,
        grid=(625,),
        in_specs=[pl.BlockSpec((128, _COUT), lambda i: (i, 0))],
        out_specs=pl.BlockSpec((8, _COUT), lambda i: (0, 0)),
        out_shape=jax.ShapeDtypeStruct((8, _COUT), jnp.float32),
    )(s)


def _final_body(s1_ref, s3_ref, st1_ref, st3_ref, gb_ref, o_ref):
    st1 = st1_ref[...]
    st3 = st3_ref[...]
    gb = gb_ref[...]
    mu1 = st1[0:1, :] / float(_NOUT)
    var1 = st1[1:2, :] / float(_NOUT) - mu1 * mu1
    a1 = gb[0:1, :] * lax.rsqrt(var1 + _EPS)
    c1 = gb[1:2, :] - mu1 * a1
    h1 = s1_ref[...] * a1 + c1
    h1 = jnp.where(h1 >= 0, h1, 0.05 * h1)
    mu3 = st3[0:1, :] / float(_NOUT)
    var3 = st3[1:2, :] / float(_NOUT) - mu3 * mu3
    a3 = gb[2:3, :] * lax.rsqrt(var3 + _EPS)
    c3 = gb[3:4, :] - mu3 * a3
    s = s3_ref[...] * a3 + c3 + h1
    o_ref[...] = jnp.where(s >= 0, s, 0.333 * s)


def _final(s1, s3, st1, st3, gbf):
    return pl.pallas_call(
        _final_body,
        grid=(_NB,),
        in_specs=[
            pl.BlockSpec((_BLK, _COUT), lambda i: (i, 0)),
            pl.BlockSpec((_BLK, _COUT), lambda i: (i, 0)),
            pl.BlockSpec((8, _COUT), lambda i: (0, 0)),
            pl.BlockSpec((8, _COUT), lambda i: (0, 0)),
            pl.BlockSpec((8, _COUT), lambda i: (0, 0)),
        ],
        out_specs=pl.BlockSpec((_BLK, _COUT), lambda i: (i, 0)),
        out_shape=jax.ShapeDtypeStruct((_NOUT, _COUT), jnp.float32),
    )(s1, s3, st1, st3, gbf)


@functools.partial(
    pl.kernel,
    mesh=_MESH,
    out_type=jax.ShapeDtypeStruct((_NPAD, _COUT), jnp.float32),
    scratch_types=[
        pltpu.VMEM((128,), jnp.int32),
        pltpu.VMEM((128, _COUT), jnp.float32),
        pltpu.SemaphoreType.DMA,
    ],
    compiler_params=pltpu.CompilerParams(use_tc_tiling_on_sc=False),
)
def _sc_skip_gather(skip_hbm, sr_hbm, out_hbm, idx_v, buf_v, sem):
    wid = lax.axis_index("s") * 2 + lax.axis_index("c")

    def chunk(c, carry):
        ch = wid * 20 + c
        base = ch * 128
        pltpu.sync_copy(sr_hbm.at[pl.ds(base, 128)], idx_v)
        pltpu.async_copy(skip_hbm.at[idx_v], buf_v, sem).wait()
        pltpu.sync_copy(buf_v, out_hbm.at[pl.ds(base, 128)])
        return carry

    lax.fori_loop(0, 20, chunk, 0)


@functools.partial(
    pl.kernel,
    mesh=_MESH,
    out_type=jax.ShapeDtypeStruct((_NPAD, _COUT), jnp.float32),
    scratch_types=[
        pltpu.VMEM((1024,), jnp.int32),
        pltpu.VMEM((1024, _COUT), jnp.float32),
        pltpu.VMEM((32, _COUT), jnp.float32),
        pltpu.SemaphoreType.DMA,
    ],
    compiler_params=pltpu.CompilerParams(use_tc_tiling_on_sc=False),
)
def _sc_gather_sum(p_hbm, cidx_hbm, out_hbm, idx_v, buf_v, acc_v, sem):
    wid = lax.axis_index("s") * 2 + lax.axis_index("c")

    def chunk(c, carry):
        ch = wid * 80 + c
        pltpu.sync_copy(cidx_hbm.at[pl.ds(ch * 1024, 1024)], idx_v)
        copies = [
            pltpu.async_copy(
                p_hbm.at[idx_v.at[pl.ds(j * 128, 128)]],
                buf_v.at[pl.ds(j * 128, 128)],
                sem,
            )
            for j in range(8)
        ]
        for cp in copies:
            cp.wait()

        def row(r, carry2):
            def tap(k, acc):
                base = r * 27 + k
                return tuple(
                    acc[l] + buf_v[base, pl.ds(16 * l, 16)] for l in range(4)
                )

            acc = lax.fori_loop(
                0, 27, tap,
                tuple(jnp.zeros((16,), jnp.float32) for _ in range(4)),
            )
            for l in range(4):
                acc_v[r, pl.ds(16 * l, 16)] = acc[l]
            return carry2

        lax.fori_loop(0, 32, row, 0)
        pltpu.sync_copy(acc_v, out_hbm.at[pl.ds(ch * 32, 32)])
        return carry

    lax.fori_loop(0, 80, chunk, 0)


def _lin(c, s):
    return (c[:, 0] * s + c[:, 1]) * s + c[:, 2]


def kernel(x, skip_features, cords, skip_cords, W_up, W1, g1, b1, W2, g2, b2,
           W3, g3, b3, spatial_size):
    s_t = 2 * spatial_size
    out_coords = (2 * cords[None, :, :] + jnp.asarray(_OFF2_NP)[:, None, :]).reshape(-1, 3)

    id_x = _lin(out_coords, s_t)
    id_s = _lin(skip_cords, s_t)
    nvox = _S2 * _S2 * _S2
    lut_s = jnp.full((nvox,), -1, jnp.int32).at[id_s].set(
        jnp.arange(_M, dtype=jnp.int32))
    r = lut_s[id_x]
    sr = jnp.where(r >= 0, r, _M)
    sr_pad = jnp.full((_NPAD,), _M, jnp.int32).at[:_NOUT].set(sr)

    lut_x = jnp.full((nvox,), -1, jnp.int32).at[id_x].set(
        jnp.arange(_NOUT, dtype=jnp.int32))
    nb = out_coords[None, :, :] + jnp.asarray(_OFF3_NP)[:, None, :]
    valid = jnp.all((nb >= 0) & (nb < _S2), axis=-1)
    nbid = (nb[..., 0] * _S2 + nb[..., 1]) * _S2 + nb[..., 2]
    nbid = jnp.clip(nbid, 0, nvox - 1)
    rows27 = jnp.where(valid, lut_x[nbid], -1)
    karr = jnp.arange(27, dtype=jnp.int32)[:, None]
    cidx = jnp.where(rows27 >= 0, karr * _TAP_ROWS + rows27, _ZROW)
    cidx_t = jnp.full((_NPAD, 27), _ZROW, jnp.int32).at[:_NOUT].set(cidx.T)
    cidx_c = cidx_t.reshape(2560, 32 * 27)
    cidx_pad = jnp.concatenate(
        [cidx_c, jnp.full((2560, 160), _ZROW, jnp.int32)], axis=1).reshape(-1)

    skip_ext = jnp.zeros((_MPAD, _COUT), jnp.float32).at[:_M].set(skip_features)
    gb1 = jnp.zeros((8, _COUT), jnp.float32).at[0].set(g1).at[1].set(b1)
    gb2 = jnp.zeros((8, _COUT), jnp.float32).at[0].set(g2).at[1].set(b2)
    gbf = (jnp.zeros((8, _COUT), jnp.float32)
           .at[0].set(g1).at[1].set(b1).at[2].set(g3).at[3].set(b3))

    skipg = _sc_skip_gather(skip_ext, sr_pad)
    feat = _upsample(x, W_up).reshape(_NOUT, _COUT)
    feat_pad = jnp.zeros((_NPAD, _COUT), jnp.float32).at[:_NOUT].set(feat)

    p1 = _p_table_first(feat_pad, skipg, W1)
    s1 = _sc_gather_sum(p1, cidx_pad)
    st1 = _stats(s1)

    p2 = _p_table_mid(s1, st1, gb1, W2)
    s2 = _sc_gather_sum(p2, cidx_pad)
    st2 = _stats(s2)

    p3 = _p_table_mid(s2, st2, gb2, W3)
    s3 = _sc_gather_sum(p3, cidx_pad)
    st3 = _stats(s3)

    return _final(s1, s3, st1, st3, gbf)
